# baseline (speedup 1.0000x reference)
import functools

import jax
import jax.numpy as jnp
from jax import lax
from jax.experimental import pallas as pl
from jax.experimental.pallas import tpu as pltpu
from jax.experimental.pallas import tpu_sc as plsc

N = 10000
E = 160000
CH = 256
H = CH // 2
P = 10240
EP = 163840
NC, NS, L = 2, 16, 16
B = 128
BC = 64
KA = EP // (NC * NS) // B
KC = EP // NS // BC
RPT = P // NS
BR = 256
GDEPTH = 1
NSLOT = GDEPTH + 1
RS = 8
SHIFT = 14

_mesh = plsc.VectorSubcoreMesh(
    core_axis_name="c", subcore_axis_name="s", num_cores=NC, num_subcores=NS
)


EPT = EP // (NC * NS)


@functools.partial(
    pl.kernel,
    out_type=jax.ShapeDtypeStruct((NC, P), jnp.float32),
    mesh=_mesh,
    scratch_types=[
        pltpu.VMEM((EPT,), jnp.int32),
        pltpu.VMEM((P,), jnp.float32),
        pltpu.VMEM((NS, RPT), jnp.float32),
        pltpu.VMEM((RPT,), jnp.float32),
        pltpu.VMEM_SHARED((NS, P), jnp.float32),
    ],
    compiler_params=pltpu.CompilerParams(needs_layout_passes=False),
)
def _deg_kernel(rows_hbm, out_hbm, idx_v, hist_v, red_v, tot_v, hist_sh):
    cid = lax.axis_index("c")
    sid = lax.axis_index("s")
    wid = cid * NS + sid
    r0 = sid * RPT
    pltpu.sync_copy(rows_hbm.at[wid], idx_v)

    zeros16 = jnp.zeros((L,), jnp.float32)

    def zbody(j, carry):
        hist_v[pl.ds(j * L, L)] = zeros16
        return carry

    lax.fori_loop(0, P // L, zbody, 0)

    ones16 = jnp.ones((L,), jnp.float32)

    def sbody(i, carry):
        idx16 = idx_v[pl.ds(i * L, L)]
        plsc.addupdate_scatter(hist_v, [idx16], ones16)
        return carry

    lax.fori_loop(0, EPT // L, sbody, 0)
    pltpu.sync_copy(hist_v, hist_sh.at[sid])
    plsc.subcore_barrier()
    for t in range(NS):
        pltpu.sync_copy(hist_sh.at[t, pl.ds(r0, RPT)], red_v.at[t])

    def rbody(c, carry):
        acc = red_v[0, pl.ds(c * L, L)]
        for t in range(1, NS):
            acc = acc + red_v[t, pl.ds(c * L, L)]
        tot_v[pl.ds(c * L, L)] = acc
        return carry

    lax.fori_loop(0, RPT // L, rbody, 0)
    pltpu.sync_copy(tot_v, out_hbm.at[cid, pl.ds(r0, RPT)])


def _mm_body(x_ref, w_ref, p_ref, y_ref, d_ref):
    deg = p_ref[0] + p_ref[1] + 1.0
    d = 1.0 / jnp.sqrt(deg)
    z = jnp.dot(x_ref[...], w_ref[...], preferred_element_type=jnp.float32)
    y_ref[0] = z * d
    d_ref[...] = d


_mm = pl.pallas_call(
    _mm_body,
    grid=(P // BR, NC),
    in_specs=[
        pl.BlockSpec((BR, CH), lambda i, c: (i, 0)),
        pl.BlockSpec((CH, H), lambda i, c: (0, c)),
        pl.BlockSpec((NC, BR, 1), lambda i, c: (0, i, 0)),
    ],
    out_specs=[
        pl.BlockSpec((1, BR, H), lambda i, c: (c, i, 0)),
        pl.BlockSpec((BR, 1), lambda i, c: (i, 0)),
    ],
    out_shape=[
        jax.ShapeDtypeStruct((NC, P, H), jnp.float32),
        jax.ShapeDtypeStruct((P, 1), jnp.float32),
    ],
)


@functools.partial(
    pl.kernel,
    out_type=jax.ShapeDtypeStruct((N, CH), jnp.float32),
    mesh=_mesh,
    scratch_types=[
        pltpu.VMEM((KC, BC), jnp.int32),
        pltpu.VMEM((RS, BC), jnp.int32),
        pltpu.VMEM((RS, BC), jnp.int32),
        pltpu.VMEM((NSLOT, BC, H), jnp.float32),
        pltpu.VMEM((RPT,), jnp.float32),
        pltpu.VMEM((H,), jnp.float32),
        pltpu.VMEM_SHARED((P, H), jnp.float32),
        pltpu.SemaphoreType.DMA((NSLOT,)),
        pltpu.SemaphoreType.DMA((NSLOT,)),
        pltpu.SemaphoreType.DMA((NSLOT,)),
    ],
)
def _prop_kernel(
    y_hbm, cr_hbm, d_hbm, b_hbm, out_hbm, cr_v, col_r, row_r, gbuf, dbuf, bbuf,
    acc_sh, gsem, gsem2, ssem
):
    cid = lax.axis_index("c")
    sid = lax.axis_index("s")
    pltpu.sync_copy(cr_hbm.at[sid], cr_v)
    r0 = sid * RPT
    pltpu.sync_copy(d_hbm.at[pl.ds(r0, RPT)], dbuf)
    pltpu.sync_copy(b_hbm.at[cid], bbuf)
    pltpu.sync_copy(y_hbm.at[cid, pl.ds(r0, RPT)], acc_sh.at[pl.ds(r0, RPT)])
    plsc.subcore_barrier()
    table = y_hbm.at[cid]

    def unpack(bi):
        rs = lax.rem(bi, RS)
        for j in range(BC // L):
            v = cr_v[bi, pl.ds(j * L, L)]
            row_r[rs, pl.ds(j * L, L)] = jnp.right_shift(v, SHIFT)
            col_r[rs, pl.ds(j * L, L)] = jnp.bitwise_and(v, (1 << SHIFT) - 1)

    HB = BC // 2

    def start_gather(bi):
        slot = lax.rem(bi, NSLOT)
        rs = lax.rem(bi, RS)
        pltpu.async_copy(
            table.at[col_r.at[rs, pl.ds(0, HB)]],
            gbuf.at[slot, pl.ds(0, HB)],
            gsem.at[slot],
        )
        pltpu.async_copy(
            table.at[col_r.at[rs, pl.ds(HB, HB)]],
            gbuf.at[slot, pl.ds(HB, HB)],
            gsem2.at[slot],
        )

    def wait_gather(bi):
        slot = lax.rem(bi, NSLOT)
        rs = lax.rem(bi, RS)
        pltpu.make_async_copy(
            table.at[col_r.at[rs, pl.ds(0, HB)]],
            gbuf.at[slot, pl.ds(0, HB)],
            gsem.at[slot],
        ).wait()
        pltpu.make_async_copy(
            table.at[col_r.at[rs, pl.ds(HB, HB)]],
            gbuf.at[slot, pl.ds(HB, HB)],
            gsem2.at[slot],
        ).wait()

    def start_scatter(bi):
        slot = lax.rem(bi, NSLOT)
        pltpu.async_copy(
            gbuf.at[slot],
            acc_sh.at[row_r.at[lax.rem(bi, RS)]],
            ssem.at[slot],
            add=True,
        )

    def wait_scatter(bi):
        slot = lax.rem(bi, NSLOT)
        pltpu.make_async_copy(
            gbuf.at[slot], acc_sh.at[row_r.at[lax.rem(bi, RS)]], ssem.at[slot]
        ).wait()

    for bi in range(GDEPTH):
        unpack(bi)
        start_gather(bi)

    def body(bi, carry):
        wait_gather(bi)
        start_scatter(bi)

        @pl.when(bi >= 1)
        def _():
            wait_scatter(bi - 1)

        @pl.when(bi + GDEPTH < KC)
        def _():
            unpack(bi + GDEPTH)
            start_gather(bi + GDEPTH)

        return carry

    lax.fori_loop(0, KC, body, 0)
    wait_scatter(KC - 1)
    plsc.subcore_barrier()

    CHK = 64

    def fin_block(lbase, nr):
        pltpu.sync_copy(
            acc_sh.at[pl.ds(r0 + lbase, nr)], gbuf.at[0, pl.ds(0, nr)]
        )

        def fgroup(g, carry):
            dvec = dbuf[pl.ds(lbase + g * L, L)]
            for rr in range(L):
                dv = dvec[rr]
                r = g * L + rr
                for j in range(H // L):
                    sl = pl.ds(j * L, L)
                    gbuf[0, r, sl] = gbuf[0, r, sl] * dv + bbuf[sl]
            return carry

        lax.fori_loop(0, nr // L, fgroup, 0)
        pltpu.sync_copy(
            gbuf.at[0, pl.ds(0, nr)],
            out_hbm.at[pl.ds(r0 + lbase, nr), pl.ds(cid * H, H)],
        )

    @pl.when(sid < NS - 1)
    def _():
        def cbody(k, carry):
            fin_block(k * CHK, CHK)
            return carry

        lax.fori_loop(0, RPT // CHK, cbody, 0)

    @pl.when(sid == NS - 1)
    def _():
        def cbody(k, carry):
            fin_block(k * CHK, CHK)
            return carry

        lax.fori_loop(0, (N - (NS - 1) * RPT) // CHK, cbody, 0)
        fin_block(6 * CHK, 16)


def kernel(x, edge_index, W, b):
    ei = edge_index.astype(jnp.int32)
    pad = jnp.full((EP - E,), N, jnp.int32)
    row = jnp.concatenate([ei[0], pad])
    col = jnp.concatenate([ei[1], pad])
    rows_a = row.reshape(NC * NS, EPT)
    cr = ((row << SHIFT) | col).reshape(NS, KC, BC)
    x_pad = jnp.pad(x, ((0, P - N), (0, 0)))
    partial = _deg_kernel(rows_a).reshape(NC, P, 1)
    y, d = _mm(x_pad, W, partial)
    return _prop_kernel(y, cr, d.reshape(P), b.reshape(NC, H))


# speedup vs baseline: 1.0397x; 1.0397x over previous
import functools

import jax
import jax.numpy as jnp
from jax import lax
from jax.experimental import pallas as pl
from jax.experimental.pallas import tpu as pltpu
from jax.experimental.pallas import tpu_sc as plsc

N = 10000
E = 160000
CH = 256
H = CH // 2
P = 10240
EP = 163840
NC, NS, L = 2, 16, 16
B = 128
BC = 64
KA = EP // (NC * NS) // B
KC = EP // NS // BC
RPT = P // NS
BR = 256
GDEPTH = 2
NSLOT = GDEPTH + 1
RS = 8
SHIFT = 14

_mesh = plsc.VectorSubcoreMesh(
    core_axis_name="c", subcore_axis_name="s", num_cores=NC, num_subcores=NS
)


EPT = EP // (NC * NS)


@functools.partial(
    pl.kernel,
    out_type=jax.ShapeDtypeStruct((NC, P), jnp.float32),
    mesh=_mesh,
    scratch_types=[
        pltpu.VMEM((EPT,), jnp.int32),
        pltpu.VMEM((P,), jnp.float32),
        pltpu.VMEM((NS, RPT), jnp.float32),
        pltpu.VMEM((RPT,), jnp.float32),
        pltpu.VMEM_SHARED((NS, P), jnp.float32),
    ],
    compiler_params=pltpu.CompilerParams(needs_layout_passes=False),
)
def _deg_kernel(rows_hbm, out_hbm, idx_v, hist_v, red_v, tot_v, hist_sh):
    cid = lax.axis_index("c")
    sid = lax.axis_index("s")
    wid = cid * NS + sid
    r0 = sid * RPT
    pltpu.sync_copy(rows_hbm.at[wid], idx_v)

    zeros16 = jnp.zeros((L,), jnp.float32)

    def zbody(j, carry):
        hist_v[pl.ds(j * L, L)] = zeros16
        return carry

    lax.fori_loop(0, P // L, zbody, 0)

    ones16 = jnp.ones((L,), jnp.float32)

    def sbody(i, carry):
        idx16 = idx_v[pl.ds(i * L, L)]
        plsc.addupdate_scatter(hist_v, [idx16], ones16)
        return carry

    lax.fori_loop(0, EPT // L, sbody, 0)
    pltpu.sync_copy(hist_v, hist_sh.at[sid])
    plsc.subcore_barrier()
    for t in range(NS):
        pltpu.sync_copy(hist_sh.at[t, pl.ds(r0, RPT)], red_v.at[t])

    def rbody(c, carry):
        acc = red_v[0, pl.ds(c * L, L)]
        for t in range(1, NS):
            acc = acc + red_v[t, pl.ds(c * L, L)]
        tot_v[pl.ds(c * L, L)] = acc
        return carry

    lax.fori_loop(0, RPT // L, rbody, 0)
    pltpu.sync_copy(tot_v, out_hbm.at[cid, pl.ds(r0, RPT)])


def _mm_body(x_ref, w_ref, p_ref, y_ref):
    deg = p_ref[0] + p_ref[1] + 1.0
    d = 1.0 / jnp.sqrt(deg)
    z = jnp.dot(x_ref[...], w_ref[...], preferred_element_type=jnp.float32)
    y_ref[0] = z * d


_mm = pl.pallas_call(
    _mm_body,
    grid=(P // BR, NC),
    in_specs=[
        pl.BlockSpec((BR, CH), lambda i, c: (i, 0)),
        pl.BlockSpec((CH, H), lambda i, c: (0, c)),
        pl.BlockSpec((NC, BR, 1), lambda i, c: (0, i, 0)),
    ],
    out_specs=pl.BlockSpec((1, BR, H), lambda i, c: (c, i, 0)),
    out_shape=jax.ShapeDtypeStruct((NC, P, H), jnp.float32),
)


@functools.partial(
    pl.kernel,
    out_type=jax.ShapeDtypeStruct((NC, P, H), jnp.float32),
    mesh=_mesh,
    scratch_types=[
        pltpu.VMEM((KC, BC), jnp.int32),
        pltpu.VMEM((RS, BC), jnp.int32),
        pltpu.VMEM((RS, BC), jnp.int32),
        pltpu.VMEM((NSLOT, BC, H), jnp.float32),
        pltpu.VMEM_SHARED((P, H), jnp.float32),
        pltpu.SemaphoreType.DMA((NSLOT,)),
        pltpu.SemaphoreType.DMA((NSLOT,)),
        pltpu.SemaphoreType.DMA((NSLOT,)),
    ],
)
def _prop_kernel(
    y_hbm, cr_hbm, out_hbm, cr_v, col_r, row_r, gbuf, acc_sh, gsem, gsem2, ssem
):
    cid = lax.axis_index("c")
    sid = lax.axis_index("s")
    pltpu.sync_copy(cr_hbm.at[sid], cr_v)
    r0 = sid * RPT
    pltpu.sync_copy(y_hbm.at[cid, pl.ds(r0, RPT)], acc_sh.at[pl.ds(r0, RPT)])
    plsc.subcore_barrier()
    table = y_hbm.at[cid]

    def unpack(bi):
        rs = lax.rem(bi, RS)
        for j in range(BC // L):
            v = cr_v[bi, pl.ds(j * L, L)]
            row_r[rs, pl.ds(j * L, L)] = jnp.right_shift(v, SHIFT)
            col_r[rs, pl.ds(j * L, L)] = jnp.bitwise_and(v, (1 << SHIFT) - 1)

    HB = BC // 2

    def start_gather(bi):
        slot = lax.rem(bi, NSLOT)
        rs = lax.rem(bi, RS)
        pltpu.async_copy(
            table.at[col_r.at[rs, pl.ds(0, HB)]],
            gbuf.at[slot, pl.ds(0, HB)],
            gsem.at[slot],
        )
        pltpu.async_copy(
            table.at[col_r.at[rs, pl.ds(HB, HB)]],
            gbuf.at[slot, pl.ds(HB, HB)],
            gsem2.at[slot],
        )

    def wait_gather(bi):
        slot = lax.rem(bi, NSLOT)
        rs = lax.rem(bi, RS)
        pltpu.make_async_copy(
            table.at[col_r.at[rs, pl.ds(0, HB)]],
            gbuf.at[slot, pl.ds(0, HB)],
            gsem.at[slot],
        ).wait()
        pltpu.make_async_copy(
            table.at[col_r.at[rs, pl.ds(HB, HB)]],
            gbuf.at[slot, pl.ds(HB, HB)],
            gsem2.at[slot],
        ).wait()

    def start_scatter(bi):
        slot = lax.rem(bi, NSLOT)
        pltpu.async_copy(
            gbuf.at[slot],
            acc_sh.at[row_r.at[lax.rem(bi, RS)]],
            ssem.at[slot],
            add=True,
        )

    def wait_scatter(bi):
        slot = lax.rem(bi, NSLOT)
        pltpu.make_async_copy(
            gbuf.at[slot], acc_sh.at[row_r.at[lax.rem(bi, RS)]], ssem.at[slot]
        ).wait()

    for bi in range(GDEPTH):
        unpack(bi)
        start_gather(bi)

    def body(bi, carry):
        @pl.when(bi + GDEPTH < KC)
        def _():
            unpack(bi + GDEPTH)

        wait_gather(bi)
        start_scatter(bi)

        @pl.when(bi >= 1)
        def _():
            wait_scatter(bi - 1)

        @pl.when(bi + GDEPTH < KC)
        def _():
            start_gather(bi + GDEPTH)

        return carry

    lax.fori_loop(0, KC, body, 0)
    wait_scatter(KC - 1)
    plsc.subcore_barrier()
    pltpu.sync_copy(acc_sh.at[pl.ds(r0, RPT)], out_hbm.at[cid, pl.ds(r0, RPT)])


def _fin_body(a_ref, p_ref, b_ref, o_ref):
    deg = p_ref[0] + p_ref[1] + 1.0
    d = 1.0 / jnp.sqrt(deg)
    o_ref[...] = jnp.concatenate([a_ref[0] * d, a_ref[1] * d], axis=1) + b_ref[...]


_fin = pl.pallas_call(
    _fin_body,
    grid=(P // BR,),
    in_specs=[
        pl.BlockSpec((NC, BR, H), lambda i: (0, i, 0)),
        pl.BlockSpec((NC, BR, 1), lambda i: (0, i, 0)),
        pl.BlockSpec((1, CH), lambda i: (0, 0)),
    ],
    out_specs=pl.BlockSpec((BR, CH), lambda i: (i, 0)),
    out_shape=jax.ShapeDtypeStruct((N, CH), jnp.float32),
)


def kernel(x, edge_index, W, b):
    ei = edge_index.astype(jnp.int32)
    pad = jnp.full((EP - E,), N, jnp.int32)
    row = jnp.concatenate([ei[0], pad])
    col = jnp.concatenate([ei[1], pad])
    rows_a = row.reshape(NC * NS, EPT)
    cr = ((row << SHIFT) | col).reshape(NS, KC, BC)
    x_pad = jnp.pad(x, ((0, P - N), (0, 0)))
    partial = _deg_kernel(rows_a).reshape(NC, P, 1)
    y = _mm(x_pad, W, partial)
    acc = _prop_kernel(y, cr)
    return _fin(acc, partial, b.reshape(1, CH))


# speedup vs baseline: 1.0408x; 1.0011x over previous
import functools

import jax
import jax.numpy as jnp
from jax import lax
from jax.experimental import pallas as pl
from jax.experimental.pallas import tpu as pltpu
from jax.experimental.pallas import tpu_sc as plsc

N = 10000
E = 160000
CH = 256
H = CH // 2
P = 10240
EP = 163840
NC, NS, L = 2, 16, 16
B = 128
BC = 80
KA = EP // (NC * NS) // B
KC = EP // NS // BC
RPT = P // NS
BR = 256
GDEPTH = 2
NSLOT = GDEPTH + 1
RS = 8
SHIFT = 14

_mesh = plsc.VectorSubcoreMesh(
    core_axis_name="c", subcore_axis_name="s", num_cores=NC, num_subcores=NS
)


EPT = EP // (NC * NS)


@functools.partial(
    pl.kernel,
    out_type=jax.ShapeDtypeStruct((NC, P), jnp.float32),
    mesh=_mesh,
    scratch_types=[
        pltpu.VMEM((EPT,), jnp.int32),
        pltpu.VMEM((P,), jnp.float32),
        pltpu.VMEM((NS, RPT), jnp.float32),
        pltpu.VMEM((RPT,), jnp.float32),
        pltpu.VMEM_SHARED((NS, P), jnp.float32),
    ],
    compiler_params=pltpu.CompilerParams(needs_layout_passes=False),
)
def _deg_kernel(rows_hbm, out_hbm, idx_v, hist_v, red_v, tot_v, hist_sh):
    cid = lax.axis_index("c")
    sid = lax.axis_index("s")
    wid = cid * NS + sid
    r0 = sid * RPT
    pltpu.sync_copy(rows_hbm.at[wid], idx_v)

    zeros16 = jnp.zeros((L,), jnp.float32)

    def zbody(j, carry):
        hist_v[pl.ds(j * L, L)] = zeros16
        return carry

    lax.fori_loop(0, P // L, zbody, 0)

    ones16 = jnp.ones((L,), jnp.float32)

    def sbody(i, carry):
        idx16 = idx_v[pl.ds(i * L, L)]
        plsc.addupdate_scatter(hist_v, [idx16], ones16)
        return carry

    lax.fori_loop(0, EPT // L, sbody, 0)
    pltpu.sync_copy(hist_v, hist_sh.at[sid])
    plsc.subcore_barrier()
    for t in range(NS):
        pltpu.sync_copy(hist_sh.at[t, pl.ds(r0, RPT)], red_v.at[t])

    def rbody(c, carry):
        acc = red_v[0, pl.ds(c * L, L)]
        for t in range(1, NS):
            acc = acc + red_v[t, pl.ds(c * L, L)]
        tot_v[pl.ds(c * L, L)] = acc
        return carry

    lax.fori_loop(0, RPT // L, rbody, 0)
    pltpu.sync_copy(tot_v, out_hbm.at[cid, pl.ds(r0, RPT)])


def _mm_body(x_ref, w_ref, p_ref, y_ref):
    deg = p_ref[0] + p_ref[1] + 1.0
    d = 1.0 / jnp.sqrt(deg)
    z = jnp.dot(x_ref[...], w_ref[...], preferred_element_type=jnp.float32)
    y_ref[0] = z * d


_mm = pl.pallas_call(
    _mm_body,
    grid=(P // BR, NC),
    in_specs=[
        pl.BlockSpec((BR, CH), lambda i, c: (i, 0)),
        pl.BlockSpec((CH, H), lambda i, c: (0, c)),
        pl.BlockSpec((NC, BR, 1), lambda i, c: (0, i, 0)),
    ],
    out_specs=pl.BlockSpec((1, BR, H), lambda i, c: (c, i, 0)),
    out_shape=jax.ShapeDtypeStruct((NC, P, H), jnp.float32),
)


@functools.partial(
    pl.kernel,
    out_type=jax.ShapeDtypeStruct((NC, P, H), jnp.float32),
    mesh=_mesh,
    scratch_types=[
        pltpu.VMEM((KC, BC), jnp.int32),
        pltpu.VMEM((RS, BC), jnp.int32),
        pltpu.VMEM((RS, BC), jnp.int32),
        pltpu.VMEM((NSLOT, BC, H), jnp.float32),
        pltpu.VMEM_SHARED((P, H), jnp.float32),
        pltpu.SemaphoreType.DMA((NSLOT,)),
        pltpu.SemaphoreType.DMA((NSLOT,)),
        pltpu.SemaphoreType.DMA((NSLOT,)),
    ],
)
def _prop_kernel(
    y_hbm, cr_hbm, out_hbm, cr_v, col_r, row_r, gbuf, acc_sh, gsem, gsem2, ssem
):
    cid = lax.axis_index("c")
    sid = lax.axis_index("s")
    pltpu.sync_copy(cr_hbm.at[sid], cr_v)
    r0 = sid * RPT
    pltpu.sync_copy(y_hbm.at[cid, pl.ds(r0, RPT)], acc_sh.at[pl.ds(r0, RPT)])
    plsc.subcore_barrier()
    table = y_hbm.at[cid]

    def unpack(bi):
        rs = lax.rem(bi, RS)
        for j in range(BC // L):
            v = cr_v[bi, pl.ds(j * L, L)]
            row_r[rs, pl.ds(j * L, L)] = jnp.right_shift(v, SHIFT)
            col_r[rs, pl.ds(j * L, L)] = jnp.bitwise_and(v, (1 << SHIFT) - 1)

    HB = BC // 2

    def start_gather(bi):
        slot = lax.rem(bi, NSLOT)
        rs = lax.rem(bi, RS)
        pltpu.async_copy(
            table.at[col_r.at[rs, pl.ds(0, HB)]],
            gbuf.at[slot, pl.ds(0, HB)],
            gsem.at[slot],
        )
        pltpu.async_copy(
            table.at[col_r.at[rs, pl.ds(HB, HB)]],
            gbuf.at[slot, pl.ds(HB, HB)],
            gsem2.at[slot],
        )

    def wait_gather(bi):
        slot = lax.rem(bi, NSLOT)
        rs = lax.rem(bi, RS)
        pltpu.make_async_copy(
            table.at[col_r.at[rs, pl.ds(0, HB)]],
            gbuf.at[slot, pl.ds(0, HB)],
            gsem.at[slot],
        ).wait()
        pltpu.make_async_copy(
            table.at[col_r.at[rs, pl.ds(HB, HB)]],
            gbuf.at[slot, pl.ds(HB, HB)],
            gsem2.at[slot],
        ).wait()

    def start_scatter(bi):
        slot = lax.rem(bi, NSLOT)
        pltpu.async_copy(
            gbuf.at[slot],
            acc_sh.at[row_r.at[lax.rem(bi, RS)]],
            ssem.at[slot],
            add=True,
        )

    def wait_scatter(bi):
        slot = lax.rem(bi, NSLOT)
        pltpu.make_async_copy(
            gbuf.at[slot], acc_sh.at[row_r.at[lax.rem(bi, RS)]], ssem.at[slot]
        ).wait()

    for bi in range(GDEPTH):
        unpack(bi)
        start_gather(bi)

    def body(bi, carry):
        @pl.when(bi + GDEPTH < KC)
        def _():
            unpack(bi + GDEPTH)

        wait_gather(bi)
        start_scatter(bi)

        @pl.when(bi >= 1)
        def _():
            wait_scatter(bi - 1)

        @pl.when(bi + GDEPTH < KC)
        def _():
            start_gather(bi + GDEPTH)

        return carry

    lax.fori_loop(0, KC, body, 0)
    wait_scatter(KC - 1)
    plsc.subcore_barrier()
    pltpu.sync_copy(acc_sh.at[pl.ds(r0, RPT)], out_hbm.at[cid, pl.ds(r0, RPT)])


def _fin_body(a_ref, p_ref, b_ref, o_ref):
    deg = p_ref[0] + p_ref[1] + 1.0
    d = 1.0 / jnp.sqrt(deg)
    o_ref[...] = jnp.concatenate([a_ref[0] * d, a_ref[1] * d], axis=1) + b_ref[...]


_fin = pl.pallas_call(
    _fin_body,
    grid=(P // BR,),
    in_specs=[
        pl.BlockSpec((NC, BR, H), lambda i: (0, i, 0)),
        pl.BlockSpec((NC, BR, 1), lambda i: (0, i, 0)),
        pl.BlockSpec((1, CH), lambda i: (0, 0)),
    ],
    out_specs=pl.BlockSpec((BR, CH), lambda i: (i, 0)),
    out_shape=jax.ShapeDtypeStruct((N, CH), jnp.float32),
)


def kernel(x, edge_index, W, b):
    ei = edge_index.astype(jnp.int32)
    pad = jnp.full((EP - E,), N, jnp.int32)
    row = jnp.concatenate([ei[0], pad])
    col = jnp.concatenate([ei[1], pad])
    rows_a = row.reshape(NC * NS, EPT)
    cr = ((row << SHIFT) | col).reshape(NS, KC, BC)
    x_pad = jnp.pad(x, ((0, P - N), (0, 0)))
    partial = _deg_kernel(rows_a).reshape(NC, P, 1)
    y = _mm(x_pad, W, partial)
    acc = _prop_kernel(y, cr)
    return _fin(acc, partial, b.reshape(1, CH))
